# baseline (device time: 99571 ns/iter reference)
import jax
import jax.numpy as jnp
from jax import lax
from jax.experimental import pallas as pl
from jax.experimental.pallas import tpu as pltpu

M = 2048
HALF = M // 2
QUAR = M // 4
K = 8192
NCH = 4
CB = M // NCH
KT = 2048
NKT = K // KT
KTA = 1024
NGA = K // KTA

_DOT_DIMS = (((1,), (1,)), ((), ()))

RS, AGX, AGY, DG = 0, 1, 2, 3


def kernel(dy, W):
    def body(dy_ref, w_ref, out_ref, a_bf16, a_stage, w_buf, acc,
             rs_stage, recv_rs, load_sems, send_sems, recv_sems):
        my_x = lax.axis_index("x")
        my_y = lax.axis_index("y")
        x_nbr = (1 - my_x, my_y)
        y_nbr = (my_x, 1 - my_y)
        row0 = my_x * HALF

        barrier_sem = pltpu.get_barrier_semaphore()
        for nbr in (x_nbr, y_nbr):
            pl.semaphore_signal(
                barrier_sem, inc=1, device_id=nbr,
                device_id_type=pl.DeviceIdType.MESH,
            )

        loc_mine = pl.ds(my_y * QUAR, QUAR)
        loc_send = pl.ds((1 - my_y) * QUAR, QUAR)
        mine = pl.ds(my_x * HALF + my_y * QUAR, QUAR)
        from_y = pl.ds(my_x * HALF + (1 - my_y) * QUAR, QUAR)

        def start_a(g):
            cp = pltpu.make_async_copy(
                dy_ref.at[pl.ds(row0, HALF), pl.ds(g * KTA, KTA)],
                a_stage.at[g % 2],
                load_sems.at[g % 2],
            )
            cp.start()
            return cp

        def start_w(t):
            c, kt = divmod(t, NKT)
            cp = pltpu.make_async_copy(
                w_ref.at[pl.ds(c * CB, CB), pl.ds(kt * KT, KT)],
                w_buf.at[t % 2],
                load_sems.at[2 + t % 2],
            )
            cp.start()
            return cp

        def remote(src, dst, stage, c, target):
            return pltpu.make_async_remote_copy(
                src_ref=src,
                dst_ref=dst,
                send_sem=send_sems.at[stage, c],
                recv_sem=recv_sems.at[stage, c],
                device_id=target,
                device_id_type=pl.DeviceIdType.MESH,
            )

        rs = [None] * NCH
        agx = [None] * NCH
        agy = [None] * NCH
        dg = [None] * NCH

        def finish_chunk(j):
            cols = pl.ds(j * CB, CB)
            rs[j].wait()
            out_ref[mine, cols] = (
                acc[j % 2, loc_mine, :]
                + recv_rs[:, cols].astype(jnp.float32)
            ).astype(jnp.bfloat16)
            agx[j] = remote(out_ref.at[mine, cols], out_ref.at[mine, cols],
                            AGX, j, x_nbr)
            agy[j] = remote(out_ref.at[mine, cols], out_ref.at[mine, cols],
                            AGY, j, y_nbr)
            agx[j].start()
            agy[j].start()

        def relay_diag(j):
            cols = pl.ds(j * CB, CB)
            agy[j].wait_recv()
            dg[j] = remote(out_ref.at[from_y, cols], out_ref.at[from_y, cols],
                           DG, j, x_nbr)
            dg[j].start()

        a_cps = [start_a(0)] + [None] * (NGA - 1)
        w_cp = [start_w(0), None]
        for c in range(NCH):
            for kt in range(NKT):
                t = c * NKT + kt
                slot = t % 2
                if c == 0:
                    for g in (2 * kt, 2 * kt + 1):
                        if g + 1 < NGA:
                            a_cps[g + 1] = start_a(g + 1)
                        a_cps[g].wait()
                        a_bf16[:, pl.ds(g * KTA, KTA)] = (
                            a_stage[g % 2].astype(jnp.bfloat16)
                        )
                if t + 1 < NCH * NKT:
                    w_cp[1 - slot] = start_w(t + 1)
                w_cp[slot].wait()
                prod = lax.dot_general(
                    a_bf16[:, pl.ds(kt * KT, KT)],
                    w_buf[slot].astype(jnp.bfloat16),
                    _DOT_DIMS,
                    preferred_element_type=jnp.float32,
                )
                if kt == 0:
                    acc[c % 2, :, :] = prod
                else:
                    acc[c % 2, :, :] = acc[c % 2, :, :] + prod

            rs_stage[c % 2, :, :] = acc[c % 2, loc_send, :].astype(jnp.bfloat16)
            if c == 0:
                pl.semaphore_wait(barrier_sem, 2)
            rs[c] = remote(rs_stage.at[c % 2],
                           recv_rs.at[:, pl.ds(c * CB, CB)], RS, c, y_nbr)
            rs[c].start()
            if c >= 1:
                finish_chunk(c - 1)
            if c >= 2:
                relay_diag(c - 2)

        last = NCH - 1
        finish_chunk(last)
        cols = pl.ds(last * CB, CB)
        dg[last] = remote(out_ref.at[mine, cols], out_ref.at[mine, cols],
                          DG, last, (1 - my_x, 1 - my_y))
        dg[last].start()
        relay_diag(NCH - 2)
        for j in range(NCH):
            agx[j].wait()
            if j == last:
                agy[j].wait()
            else:
                agy[j].wait_send()
            dg[j].wait()

    return pl.pallas_call(
        body,
        out_shape=jax.ShapeDtypeStruct((M, M), jnp.bfloat16),
        in_specs=[
            pl.BlockSpec(memory_space=pl.ANY),
            pl.BlockSpec(memory_space=pl.ANY),
        ],
        out_specs=pl.BlockSpec(memory_space=pltpu.VMEM),
        scratch_shapes=[
            pltpu.VMEM((HALF, K), jnp.bfloat16),
            pltpu.VMEM((2, HALF, KTA), jnp.float32),
            pltpu.VMEM((2, CB, KT), jnp.float32),
            pltpu.VMEM((2, HALF, CB), jnp.float32),
            pltpu.VMEM((2, QUAR, CB), jnp.bfloat16),
            pltpu.VMEM((QUAR, M), jnp.bfloat16),
            pltpu.SemaphoreType.DMA((4,)),
            pltpu.SemaphoreType.DMA((4, NCH)),
            pltpu.SemaphoreType.DMA((4, NCH)),
        ],
        compiler_params=pltpu.CompilerParams(
            collective_id=0, vmem_limit_bytes=64 * 1024 * 1024,
        ),
    )(dy, W)


# device time: 97009 ns/iter; 1.0264x vs baseline; 1.0264x over previous
import jax
import jax.numpy as jnp
from jax import lax
from jax.experimental import pallas as pl
from jax.experimental.pallas import tpu as pltpu

M = 2048
HALF = M // 2
QUAR = M // 4
K = 8192
NCH = 4
CB = M // NCH
KT = 2048
NKT = K // KT

_DOT_DIMS = (((1,), (1,)), ((), ()))

RS, AGX, AGY, DG = 0, 1, 2, 3


def kernel(dy, W):
    def body(dy_ref, w_ref, out_ref, a_bf16, a_stage, w_buf, acc,
             rs_stage, recv_rs, load_sems, send_sems, recv_sems):
        my_x = lax.axis_index("x")
        my_y = lax.axis_index("y")
        x_nbr = (1 - my_x, my_y)
        y_nbr = (my_x, 1 - my_y)
        row0 = my_x * HALF

        barrier_sem = pltpu.get_barrier_semaphore()
        for nbr in (x_nbr, y_nbr):
            pl.semaphore_signal(
                barrier_sem, inc=1, device_id=nbr,
                device_id_type=pl.DeviceIdType.MESH,
            )

        loc_mine = pl.ds(my_y * QUAR, QUAR)
        loc_send = pl.ds((1 - my_y) * QUAR, QUAR)
        mine = pl.ds(my_x * HALF + my_y * QUAR, QUAR)
        from_y = pl.ds(my_x * HALF + (1 - my_y) * QUAR, QUAR)

        def start_a(kt):
            cp = pltpu.make_async_copy(
                dy_ref.at[pl.ds(row0, HALF), pl.ds(kt * KT, KT)],
                a_stage,
                load_sems.at[0],
            )
            cp.start()
            return cp

        def start_w(t):
            c, kt = divmod(t, NKT)
            cp = pltpu.make_async_copy(
                w_ref.at[pl.ds(c * CB, CB), pl.ds(kt * KT, KT)],
                w_buf.at[t % 2],
                load_sems.at[1 + t % 2],
            )
            cp.start()
            return cp

        def remote(src, dst, stage, c, target):
            return pltpu.make_async_remote_copy(
                src_ref=src,
                dst_ref=dst,
                send_sem=send_sems.at[stage, c],
                recv_sem=recv_sems.at[stage, c],
                device_id=target,
                device_id_type=pl.DeviceIdType.MESH,
            )

        rs = [None] * NCH
        agx = [None] * NCH
        agy = [None] * NCH
        dg = [None] * NCH

        def finish_chunk(j):
            cols = pl.ds(j * CB, CB)
            rs[j].wait()
            out_ref[mine, cols] = (
                acc[j % 2, loc_mine, :]
                + recv_rs[:, cols].astype(jnp.float32)
            ).astype(jnp.bfloat16)
            agx[j] = remote(out_ref.at[mine, cols], out_ref.at[mine, cols],
                            AGX, j, x_nbr)
            agy[j] = remote(out_ref.at[mine, cols], out_ref.at[mine, cols],
                            AGY, j, y_nbr)
            agx[j].start()
            agy[j].start()

        def relay_diag(j):
            cols = pl.ds(j * CB, CB)
            agy[j].wait_recv()
            dg[j] = remote(out_ref.at[from_y, cols], out_ref.at[from_y, cols],
                           DG, j, x_nbr)
            dg[j].start()

        a_cp = start_a(0)
        w_cp = [start_w(0), None]
        for c in range(NCH):
            for kt in range(NKT):
                t = c * NKT + kt
                slot = t % 2
                if c == 0:
                    a_cp.wait()
                    a_bf16[:, pl.ds(kt * KT, KT)] = (
                        a_stage[...].astype(jnp.bfloat16)
                    )
                    if kt + 1 < NKT:
                        a_cp = start_a(kt + 1)
                if t + 1 < NCH * NKT:
                    w_cp[1 - slot] = start_w(t + 1)
                w_cp[slot].wait()
                prod = lax.dot_general(
                    a_bf16[:, pl.ds(kt * KT, KT)],
                    w_buf[slot].astype(jnp.bfloat16),
                    _DOT_DIMS,
                    preferred_element_type=jnp.float32,
                )
                if kt == 0:
                    acc[c % 2, :, :] = prod
                else:
                    acc[c % 2, :, :] = acc[c % 2, :, :] + prod

            rs_stage[c % 2, :, :] = acc[c % 2, loc_send, :].astype(jnp.bfloat16)
            if c == 0:
                pl.semaphore_wait(barrier_sem, 2)
            rs[c] = remote(rs_stage.at[c % 2],
                           recv_rs.at[:, pl.ds(c * CB, CB)], RS, c, y_nbr)
            rs[c].start()
            if c >= 1:
                finish_chunk(c - 1)
            if c >= 2:
                relay_diag(c - 2)

        finish_chunk(NCH - 1)
        relay_diag(NCH - 2)
        relay_diag(NCH - 1)
        for j in range(NCH):
            agx[j].wait()
            agy[j].wait_send()
            dg[j].wait()

    return pl.pallas_call(
        body,
        out_shape=jax.ShapeDtypeStruct((M, M), jnp.bfloat16),
        in_specs=[
            pl.BlockSpec(memory_space=pl.ANY),
            pl.BlockSpec(memory_space=pl.ANY),
        ],
        out_specs=pl.BlockSpec(memory_space=pltpu.VMEM),
        scratch_shapes=[
            pltpu.VMEM((HALF, K), jnp.bfloat16),
            pltpu.VMEM((HALF, KT), jnp.float32),
            pltpu.VMEM((2, CB, KT), jnp.float32),
            pltpu.VMEM((2, HALF, CB), jnp.float32),
            pltpu.VMEM((2, QUAR, CB), jnp.bfloat16),
            pltpu.VMEM((QUAR, M), jnp.bfloat16),
            pltpu.SemaphoreType.DMA((3,)),
            pltpu.SemaphoreType.DMA((4, NCH)),
            pltpu.SemaphoreType.DMA((4, NCH)),
        ],
        compiler_params=pltpu.CompilerParams(
            collective_id=0, vmem_limit_bytes=64 * 1024 * 1024,
        ),
    )(dy, W)
